# baseline (speedup 1.0000x reference)
import dataclasses
import functools

import numpy as np
import jax
import jax.numpy as jnp
from jax import lax
from jax.experimental import pallas as pl
from jax.experimental.pallas import tpu as pltpu
from jax.experimental.pallas import tpu_sc as plsc

_DIM = 33
_N3 = _DIM * _DIM * _DIM
_FLAT = 3 * _N3
_CPAD = 36224
_PAD = 3 * _CPAD
_B = 4
_H = _W = 512
_HW = _H * _W
_FH = _FW = 32
_FHW = _FH * _FW
_C_IN = 1024
_C_MID = 256
_NLUT = 6
_CH = 1024
_NTEC = 32
_PARTS = _NTEC // _B
_PER_TEC = _HW // _PARTS
_NCHUNK = _PER_TEC // _CH
_INV_BIN = np.float32((_DIM - 1) / 1.000001)


def _resize_mat():
    ys = np.linspace(0.0, float(_H - 1), _FH)
    y0 = np.floor(ys).astype(np.int64)
    y1 = np.clip(y0 + 1, 0, _H - 1)
    wy = (ys - y0).astype(np.float32)
    r = np.zeros((_FH, _H), np.float32)
    r[np.arange(_FH), y0] += 1.0 - wy
    r[np.arange(_FH), y1] += wy
    return r


_RY = _resize_mat()
_EH = np.kron(np.eye(_FH, dtype=np.float32),
              np.ones((_FW, 1), np.float32))
_RXE = np.tile(_RY, (_FH, 1)).astype(np.float32)

_DOT = functools.partial(jnp.dot, precision=lax.Precision.HIGHEST)


def _head_body(fea_ref, maskr_ref, ry_ref, eh_ref, rxe_ref, cw_ref, cb_ref,
               fcw_ref, fcb_ref, l0_ref, l1_ref, l2_ref, l3_ref, l4_ref,
               l5_ref, out_ref):
    ry = ry_ref[...]
    eh = eh_ref[...]
    rxe = rxe_ref[...]
    cw = cw_ref[...]
    cb = cb_ref[...].reshape(_C_MID, 1)
    fcw = fcw_ref[...]
    fcb = fcb_ref[...].reshape(_NLUT, 1)

    eh_bf = eh.astype(jnp.bfloat16)
    masks_wide = jnp.concatenate([maskr_ref[b] for b in range(_B)], axis=1)
    q_all = _DOT(ry, masks_wide)
    dmfs = []
    for b in range(_B):
        q = q_all[:, b * _W:(b + 1) * _W]
        ehq = jnp.dot(eh_bf, q.astype(jnp.bfloat16),
                      preferred_element_type=jnp.float32)
        dmf = jnp.sum(ehq * rxe, axis=1, keepdims=True)
        dmfs.append(dmf)

    ones_col = jnp.ones((_FHW, 1), jnp.float32)
    dm5 = jnp.concatenate(dmfs + [ones_col], axis=1)
    s_all = jnp.dot(fea_ref[...], dm5.astype(jnp.bfloat16),
                    preferred_element_type=jnp.float32)

    cols = []
    cnts = []
    for b in range(_B):
        s_fg = s_all[b * _C_IN:(b + 1) * _C_IN, b:b + 1]
        s_full = s_all[b * _C_IN:(b + 1) * _C_IN, _B:_B + 1]
        s_bg = s_full - s_fg
        cols += [s_fg, s_bg, s_full]
        m_fg = jnp.sum(dmfs[b], axis=0, keepdims=True)
        m_bg = jnp.sum(1.0 - dmfs[b], axis=0, keepdims=True)
        cnts += [m_fg, m_bg, m_fg + m_bg]
    sp = jnp.concatenate(cols, axis=1)
    cnt = jnp.concatenate(cnts, axis=1)

    feat = _DOT(cw, sp)
    feat = (feat + cb * cnt) / (cnt + 1e-6)

    logits = fcb
    for j in range(3):
        feat_j = jnp.concatenate(
            [feat[:, 3 * b + j:3 * b + j + 1] for b in range(_B)], axis=1)
        logits = logits + _DOT(fcw[:, j * _C_MID:(j + 1) * _C_MID], feat_j)
    z = logits - jnp.max(logits, axis=0, keepdims=True)
    e = jnp.exp(z)
    p_all = e / jnp.sum(e, axis=0, keepdims=True)
    lrefs = (l0_ref, l1_ref, l2_ref, l3_ref, l4_ref, l5_ref)
    for b in range(_B):
        acc = lrefs[0][...] * p_all[0:1, b:b + 1]
        for n in range(1, _NLUT):
            acc = acc + lrefs[n][...] * p_all[n:n + 1, b:b + 1]
        for c in range(3):
            out_ref[b:b + 1, c * _CPAD:c * _CPAD + _N3] = acc[c:c + 1, :]


_head = pl.pallas_call(
    _head_body,
    out_shape=jax.ShapeDtypeStruct((_B, _PAD), jnp.float32),
    compiler_params=pltpu.CompilerParams(vmem_limit_bytes=63 * 1024 * 1024),
)


_SC_PARAMS = pltpu.CompilerParams()
if "needs_layout_passes" in pltpu.CompilerParams.__dataclass_fields__:
    _SC_PARAMS = dataclasses.replace(_SC_PARAMS, needs_layout_passes=False)


_ROWS_PER_TEC = _H // _PARTS
_RCH = 2


@functools.partial(
    pl.kernel,
    out_type=jax.ShapeDtypeStruct((_B, 3, _H, _W), jnp.float32),
    mesh=plsc.VectorSubcoreMesh(core_axis_name="c", subcore_axis_name="s"),
    compiler_params=_SC_PARAMS,
    scratch_types=[
        pltpu.VMEM((_PAD,), jnp.float32),
        pltpu.VMEM((2, 3, _RCH, _W), jnp.float32),
        pltpu.VMEM((2, _RCH, _W), jnp.float32),
        pltpu.VMEM((2, 3, _RCH, _W), jnp.float32),
        pltpu.SemaphoreType.DMA((2,)),
        pltpu.SemaphoreType.DMA((2,)),
        pltpu.SemaphoreType.DMA((2,)),
    ],
)
def _sc_apply(img_hbm, mask_hbm, mlut_hbm, out_hbm, lut_v, in_v, m_v, o_v,
              sem_i, sem_m, sem_o):
    wid = lax.axis_index("s") * 2 + lax.axis_index("c")
    batch = wid // _PARTS
    row0 = (wid % _PARTS) * _ROWS_PER_TEC

    pltpu.sync_copy(mlut_hbm.at[batch], lut_v)

    offs = (1, _DIM, _DIM + 1,
            _DIM * _DIM, _DIM * _DIM + 1, _DIM * _DIM + _DIM,
            _DIM * _DIM + _DIM + 1)

    def in_copies(g, slot):
        row = row0 + g * _RCH
        return (
            pltpu.make_async_copy(
                img_hbm.at[batch, :, pl.ds(row, _RCH), :],
                in_v.at[slot], sem_i.at[slot]),
            pltpu.make_async_copy(
                mask_hbm.at[batch, 0, pl.ds(row, _RCH), :],
                m_v.at[slot], sem_m.at[slot]),
        )

    def out_copy(g, slot):
        row = row0 + g * _RCH
        return pltpu.make_async_copy(
            o_v.at[slot], out_hbm.at[batch, :, pl.ds(row, _RCH), :],
            sem_o.at[slot])

    def compute_row(slot, r2):
        @plsc.parallel_loop(0, _W, 16, unroll=2)
        def _px(i):
            sl = pl.ds(i, 16)
            rgb = (in_v[slot, 0, r2, sl], in_v[slot, 1, r2, sl],
                   in_v[slot, 2, r2, sl])
            m = m_v[slot, r2, sl]
            rs = rgb[0] * _INV_BIN
            gs = rgb[1] * _INV_BIN
            bs = rgb[2] * _INV_BIN
            ri = rs.astype(jnp.int32)
            gi = gs.astype(jnp.int32)
            bi = bs.astype(jnp.int32)
            fr = rs - ri.astype(jnp.float32)
            fg = gs - gi.astype(jnp.float32)
            fb = bs - bi.astype(jnp.float32)
            wr0 = 1.0 - fr
            wg0 = 1.0 - fg
            wb0 = 1.0 - fb
            w00 = wr0 * wg0
            w10 = fr * wg0
            w01 = wr0 * fg
            w11 = fr * fg
            w = (w00 * wb0, w10 * wb0, w01 * wb0, w11 * wb0,
                 w00 * fb, w10 * fb, w01 * fb, w11 * fb)
            idx0 = ri + gi * _DIM + bi * (_DIM * _DIM)
            idxs = (idx0,) + tuple(idx0 + o for o in offs)
            one_m = 1.0 - m
            for c in range(3):
                plane = lut_v.at[pl.ds(c * _CPAD, _CPAD)]
                v = [plsc.load_gather(plane, [ix]) for ix in idxs]
                acc = (((v[0] * w[0] + v[1] * w[1])
                        + (v[2] * w[2] + v[3] * w[3]))
                       + ((v[4] * w[4] + v[5] * w[5])
                          + (v[6] * w[6] + v[7] * w[7])))
                o_v[slot, c, r2, sl] = acc * m + rgb[c] * one_m

    for c in in_copies(0, 0) + in_copies(1, 1):
        c.start()

    @pl.loop(0, _NCHUNK, step=2)
    def _chunk(g):
        for slot in (0, 1):
            gg = g + slot
            for c in in_copies(gg, slot):
                c.wait()

            @pl.when(gg >= 2)
            def _():
                out_copy(gg - 2, slot).wait()

            for r2 in range(_RCH):
                compute_row(slot, r2)
            out_copy(gg, slot).start()

            @pl.when(gg + 2 < _NCHUNK)
            def _():
                for c in in_copies(gg + 2, slot):
                    c.start()

    out_copy(_NCHUNK - 2, 0).wait()
    out_copy(_NCHUNK - 1, 1).wait()


def kernel(encoder_outputs, image, mask, conv_w, conv_b, fc_w, fc_b,
           lut0, lut1, lut2, lut3, lut4, lut5):
    fea = encoder_outputs.reshape(_B * _C_IN, _FHW).astype(jnp.bfloat16)
    maskr = mask.reshape(_B, _H, _W)
    flat6 = [l.reshape(3, _N3) for l in (lut0, lut1, lut2, lut3, lut4, lut5)]

    mlut = _head(fea, maskr, jnp.asarray(_RY), jnp.asarray(_EH),
                 jnp.asarray(_RXE), conv_w, conv_b, fc_w, fc_b, *flat6)

    return _sc_apply(image, mask, mlut)


# speedup vs baseline: 1.0264x; 1.0264x over previous
import dataclasses
import functools

import numpy as np
import jax
import jax.numpy as jnp
from jax import lax
from jax.experimental import pallas as pl
from jax.experimental.pallas import tpu as pltpu
from jax.experimental.pallas import tpu_sc as plsc

_DIM = 33
_N3 = _DIM * _DIM * _DIM
_FLAT = 3 * _N3
_CPAD = 36224
_PAD = 3 * _CPAD
_B = 4
_H = _W = 512
_HW = _H * _W
_FH = _FW = 32
_FHW = _FH * _FW
_C_IN = 1024
_C_MID = 256
_NLUT = 6
_CH = 1024
_NTEC = 32
_PARTS = _NTEC // _B
_PER_TEC = _HW // _PARTS
_NCHUNK = _PER_TEC // _CH
_INV_BIN = np.float32((_DIM - 1) / 1.000001)


def _resize_mat():
    ys = np.linspace(0.0, float(_H - 1), _FH)
    y0 = np.floor(ys).astype(np.int64)
    y1 = np.clip(y0 + 1, 0, _H - 1)
    wy = (ys - y0).astype(np.float32)
    r = np.zeros((_FH, _H), np.float32)
    r[np.arange(_FH), y0] += 1.0 - wy
    r[np.arange(_FH), y1] += wy
    return r


_RY = _resize_mat()
_EH = np.kron(np.eye(_FH, dtype=np.float32),
              np.ones((_FW, 1), np.float32))
_RXE = np.tile(_RY, (_FH, 1)).astype(np.float32)

_DOT = functools.partial(jnp.dot, precision=lax.Precision.HIGHEST)


def _head_body(fea_ref, maskr_ref, ry_ref, eh_ref, rxe_ref, cw_ref, cb_ref,
               fcw_ref, fcb_ref, l0_ref, l1_ref, l2_ref, l3_ref, l4_ref,
               l5_ref, out_ref):
    ry = ry_ref[...]
    eh = eh_ref[...]
    rxe = rxe_ref[...]
    cw = cw_ref[...]
    cb = cb_ref[...].reshape(_C_MID, 1)
    fcw = fcw_ref[...]
    fcb = fcb_ref[...].reshape(_NLUT, 1)

    eh_bf = eh.astype(jnp.bfloat16)
    masks_wide = jnp.concatenate([maskr_ref[b] for b in range(_B)], axis=1)
    q_all = _DOT(ry, masks_wide)
    dmfs = []
    for b in range(_B):
        q = q_all[:, b * _W:(b + 1) * _W]
        ehq = jnp.dot(eh_bf, q.astype(jnp.bfloat16),
                      preferred_element_type=jnp.float32)
        dmf = jnp.sum(ehq * rxe, axis=1, keepdims=True)
        dmfs.append(dmf)

    ones_col = jnp.ones((_FHW, 1), jnp.float32)
    dm5 = jnp.concatenate(dmfs + [ones_col], axis=1)
    s_all = jnp.dot(fea_ref[...], dm5.astype(jnp.bfloat16),
                    preferred_element_type=jnp.float32)

    cols = []
    cnts = []
    for b in range(_B):
        s_fg = s_all[b * _C_IN:(b + 1) * _C_IN, b:b + 1]
        s_full = s_all[b * _C_IN:(b + 1) * _C_IN, _B:_B + 1]
        s_bg = s_full - s_fg
        cols += [s_fg, s_bg, s_full]
        m_fg = jnp.sum(dmfs[b], axis=0, keepdims=True)
        m_bg = jnp.sum(1.0 - dmfs[b], axis=0, keepdims=True)
        cnts += [m_fg, m_bg, m_fg + m_bg]
    sp = jnp.concatenate(cols, axis=1)
    cnt = jnp.concatenate(cnts, axis=1)

    feat = _DOT(cw, sp)
    feat = (feat + cb * cnt) / (cnt + 1e-6)

    logits = fcb
    for j in range(3):
        feat_j = jnp.concatenate(
            [feat[:, 3 * b + j:3 * b + j + 1] for b in range(_B)], axis=1)
        logits = logits + _DOT(fcw[:, j * _C_MID:(j + 1) * _C_MID], feat_j)
    z = logits - jnp.max(logits, axis=0, keepdims=True)
    e = jnp.exp(z)
    p_all = e / jnp.sum(e, axis=0, keepdims=True)
    lrefs = (l0_ref, l1_ref, l2_ref, l3_ref, l4_ref, l5_ref)
    for b in range(_B):
        acc = lrefs[0][...] * p_all[0:1, b:b + 1]
        for n in range(1, _NLUT):
            acc = acc + lrefs[n][...] * p_all[n:n + 1, b:b + 1]
        for c in range(3):
            out_ref[b:b + 1, c * _CPAD:c * _CPAD + _N3] = acc[c:c + 1, :]


_head = pl.pallas_call(
    _head_body,
    out_shape=jax.ShapeDtypeStruct((_B, _PAD), jnp.float32),
    compiler_params=pltpu.CompilerParams(vmem_limit_bytes=63 * 1024 * 1024),
)


_SC_PARAMS = pltpu.CompilerParams()
if "needs_layout_passes" in pltpu.CompilerParams.__dataclass_fields__:
    _SC_PARAMS = dataclasses.replace(_SC_PARAMS, needs_layout_passes=False)


_ROWS_PER_TEC = _H // _PARTS
_RCH = 2


@functools.partial(
    pl.kernel,
    out_type=jax.ShapeDtypeStruct((_B, 3, _H, _W), jnp.float32),
    mesh=plsc.VectorSubcoreMesh(core_axis_name="c", subcore_axis_name="s"),
    compiler_params=_SC_PARAMS,
    scratch_types=[
        pltpu.VMEM((_PAD,), jnp.float32),
        pltpu.VMEM((2, 3, _CH), jnp.float32),
        pltpu.VMEM((2, _CH), jnp.float32),
        pltpu.VMEM((2, 3, _CH), jnp.float32),
        pltpu.SemaphoreType.DMA((2,)),
        pltpu.SemaphoreType.DMA((2,)),
        pltpu.SemaphoreType.DMA((2,)),
    ],
)
def _sc_apply(img_hbm, mask_hbm, mlut_hbm, out_hbm, lut_v, in_v, m_v, o_v,
              sem_i, sem_m, sem_o):
    wid = lax.axis_index("s") * 2 + lax.axis_index("c")
    batch = wid // _PARTS
    row0 = (wid % _PARTS) * _ROWS_PER_TEC

    pltpu.sync_copy(mlut_hbm.at[batch], lut_v)

    offs = (1, _DIM, _DIM + 1,
            _DIM * _DIM, _DIM * _DIM + 1, _DIM * _DIM + _DIM,
            _DIM * _DIM + _DIM + 1)

    def in_copies(g, slot):
        row = row0 + g * _RCH
        cps = []
        for r in range(_RCH):
            cps.append(pltpu.make_async_copy(
                img_hbm.at[batch, :, row + r, :],
                in_v.at[slot, :, pl.ds(r * _W, _W)], sem_i.at[slot]))
            cps.append(pltpu.make_async_copy(
                mask_hbm.at[batch, 0, row + r, :],
                m_v.at[slot, pl.ds(r * _W, _W)], sem_m.at[slot]))
        return tuple(cps)

    def out_copy(g, slot):
        row = row0 + g * _RCH
        return tuple(
            pltpu.make_async_copy(
                o_v.at[slot, :, pl.ds(r * _W, _W)],
                out_hbm.at[batch, :, row + r, :], sem_o.at[slot])
            for r in range(_RCH))

    def compute(slot):
        @plsc.parallel_loop(0, _CH, 16, unroll=2)
        def _px(i):
            sl = pl.ds(i, 16)
            rgb = (in_v[slot, 0, sl], in_v[slot, 1, sl], in_v[slot, 2, sl])
            m = m_v[slot, sl]
            rs = rgb[0] * _INV_BIN
            gs = rgb[1] * _INV_BIN
            bs = rgb[2] * _INV_BIN
            ri = rs.astype(jnp.int32)
            gi = gs.astype(jnp.int32)
            bi = bs.astype(jnp.int32)
            fr = rs - ri.astype(jnp.float32)
            fg = gs - gi.astype(jnp.float32)
            fb = bs - bi.astype(jnp.float32)
            wr0 = 1.0 - fr
            wg0 = 1.0 - fg
            wb0 = 1.0 - fb
            w00 = wr0 * wg0
            w10 = fr * wg0
            w01 = wr0 * fg
            w11 = fr * fg
            w = (w00 * wb0, w10 * wb0, w01 * wb0, w11 * wb0,
                 w00 * fb, w10 * fb, w01 * fb, w11 * fb)
            idx0 = ri + gi * _DIM + bi * (_DIM * _DIM)
            idxs = (idx0,) + tuple(idx0 + o for o in offs)
            one_m = 1.0 - m
            for c in range(3):
                plane = lut_v.at[pl.ds(c * _CPAD, _CPAD)]
                v = [plsc.load_gather(plane, [ix]) for ix in idxs]
                acc = (((v[0] * w[0] + v[1] * w[1])
                        + (v[2] * w[2] + v[3] * w[3]))
                       + ((v[4] * w[4] + v[5] * w[5])
                          + (v[6] * w[6] + v[7] * w[7])))
                o_v[slot, c, sl] = acc * m + rgb[c] * one_m

    for c in in_copies(0, 0) + in_copies(1, 1):
        c.start()

    @pl.loop(0, _NCHUNK, step=2)
    def _chunk(g):
        for slot in (0, 1):
            gg = g + slot
            for c in in_copies(gg, slot):
                c.wait()

            @pl.when(gg >= 2)
            def _():
                for c in out_copy(gg - 2, slot):
                    c.wait()

            compute(slot)
            for c in out_copy(gg, slot):
                c.start()

            @pl.when(gg + 2 < _NCHUNK)
            def _():
                for c in in_copies(gg + 2, slot):
                    c.start()

    for c in out_copy(_NCHUNK - 2, 0) + out_copy(_NCHUNK - 1, 1):
        c.wait()


def kernel(encoder_outputs, image, mask, conv_w, conv_b, fc_w, fc_b,
           lut0, lut1, lut2, lut3, lut4, lut5):
    fea = encoder_outputs.reshape(_B * _C_IN, _FHW).astype(jnp.bfloat16)
    maskr = mask.reshape(_B, _H, _W)
    flat6 = [l.reshape(3, _N3) for l in (lut0, lut1, lut2, lut3, lut4, lut5)]

    mlut = _head(fea, maskr, jnp.asarray(_RY), jnp.asarray(_EH),
                 jnp.asarray(_RXE), conv_w, conv_b, fc_w, fc_b, *flat6)

    return _sc_apply(image, mask, mlut)


# speedup vs baseline: 1.1071x; 1.0786x over previous
import dataclasses
import functools

import numpy as np
import jax
import jax.numpy as jnp
from jax import lax
from jax.experimental import pallas as pl
from jax.experimental.pallas import tpu as pltpu
from jax.experimental.pallas import tpu_sc as plsc

_DIM = 33
_N3 = _DIM * _DIM * _DIM
_FLAT = 3 * _N3
_CPAD = 36224
_PAD = 3 * _CPAD
_B = 4
_H = _W = 512
_HW = _H * _W
_FH = _FW = 32
_FHW = _FH * _FW
_C_IN = 1024
_C_MID = 256
_NLUT = 6
_CH = 1024
_NTEC = 32
_PARTS = _NTEC // _B
_PER_TEC = _HW // _PARTS
_NCHUNK = _PER_TEC // _CH
_INV_BIN = np.float32((_DIM - 1) / 1.000001)


def _resize_mat():
    ys = np.linspace(0.0, float(_H - 1), _FH)
    y0 = np.floor(ys).astype(np.int64)
    y1 = np.clip(y0 + 1, 0, _H - 1)
    wy = (ys - y0).astype(np.float32)
    r = np.zeros((_FH, _H), np.float32)
    r[np.arange(_FH), y0] += 1.0 - wy
    r[np.arange(_FH), y1] += wy
    return r


_RY = _resize_mat()
_EH = np.kron(np.eye(_FH, dtype=np.float32),
              np.ones((_FW, 1), np.float32))
_RXE = np.tile(_RY, (_FH, 1)).astype(np.float32)

_DOT = functools.partial(jnp.dot, precision=lax.Precision.HIGHEST)


def _head_body(fea_ref, maskr_ref, ry_ref, eh_ref, rxe_ref, cw_ref, cb_ref,
               fcw_ref, fcb_ref, l0_ref, l1_ref, l2_ref, l3_ref, l4_ref,
               l5_ref, out_ref):
    ry = ry_ref[...]
    eh = eh_ref[...]
    rxe = rxe_ref[...]
    cw = cw_ref[...]
    cb = cb_ref[...].reshape(_C_MID, 1)
    fcw = fcw_ref[...]
    fcb = fcb_ref[...].reshape(_NLUT, 1)

    eh_bf = eh.astype(jnp.bfloat16)
    masks_wide = jnp.concatenate([maskr_ref[b] for b in range(_B)], axis=1)
    q_all = _DOT(ry, masks_wide)
    dmfs = []
    for b in range(_B):
        q = q_all[:, b * _W:(b + 1) * _W]
        ehq = jnp.dot(eh_bf, q.astype(jnp.bfloat16),
                      preferred_element_type=jnp.float32)
        dmf = jnp.sum(ehq * rxe, axis=1, keepdims=True)
        dmfs.append(dmf)

    ones_col = jnp.ones((_FHW, 1), jnp.float32)
    dm5 = jnp.concatenate(dmfs + [ones_col], axis=1)
    s_all = jnp.dot(fea_ref[...], dm5.astype(jnp.bfloat16),
                    preferred_element_type=jnp.float32)

    cols = []
    cnts = []
    for b in range(_B):
        s_fg = s_all[b * _C_IN:(b + 1) * _C_IN, b:b + 1]
        s_full = s_all[b * _C_IN:(b + 1) * _C_IN, _B:_B + 1]
        s_bg = s_full - s_fg
        cols += [s_fg, s_bg, s_full]
        m_fg = jnp.sum(dmfs[b], axis=0, keepdims=True)
        m_bg = jnp.sum(1.0 - dmfs[b], axis=0, keepdims=True)
        cnts += [m_fg, m_bg, m_fg + m_bg]
    sp = jnp.concatenate(cols, axis=1)
    cnt = jnp.concatenate(cnts, axis=1)

    feat = _DOT(cw, sp)
    feat = (feat + cb * cnt) / (cnt + 1e-6)

    logits = fcb
    for j in range(3):
        feat_j = jnp.concatenate(
            [feat[:, 3 * b + j:3 * b + j + 1] for b in range(_B)], axis=1)
        logits = logits + _DOT(fcw[:, j * _C_MID:(j + 1) * _C_MID], feat_j)
    z = logits - jnp.max(logits, axis=0, keepdims=True)
    e = jnp.exp(z)
    p_all = e / jnp.sum(e, axis=0, keepdims=True)
    lrefs = (l0_ref, l1_ref, l2_ref, l3_ref, l4_ref, l5_ref)
    for b in range(_B):
        acc = lrefs[0][...] * p_all[0:1, b:b + 1]
        for n in range(1, _NLUT):
            acc = acc + lrefs[n][...] * p_all[n:n + 1, b:b + 1]
        for c in range(3):
            out_ref[b:b + 1, c * _CPAD:c * _CPAD + _N3] = acc[c:c + 1, :]


_head = pl.pallas_call(
    _head_body,
    out_shape=jax.ShapeDtypeStruct((_B, _PAD), jnp.float32),
    compiler_params=pltpu.CompilerParams(vmem_limit_bytes=63 * 1024 * 1024),
)


_SC_PARAMS = pltpu.CompilerParams()
if "needs_layout_passes" in pltpu.CompilerParams.__dataclass_fields__:
    _SC_PARAMS = dataclasses.replace(_SC_PARAMS, needs_layout_passes=False)


_ROWS_PER_TEC = _H // _PARTS
_RCH = 2


@functools.partial(
    pl.kernel,
    out_type=jax.ShapeDtypeStruct((_B, 3, _H, _W), jnp.float32),
    mesh=plsc.VectorSubcoreMesh(core_axis_name="c", subcore_axis_name="s"),
    compiler_params=_SC_PARAMS,
    scratch_types=[
        pltpu.VMEM((_PAD,), jnp.float32),
        pltpu.VMEM((2, 3, _RCH, _W), jnp.float32),
        pltpu.VMEM((2, _RCH, _W), jnp.float32),
        pltpu.VMEM((2, 3, _RCH, _W), jnp.float32),
        pltpu.SemaphoreType.DMA((2,)),
        pltpu.SemaphoreType.DMA((2,)),
        pltpu.SemaphoreType.DMA((2,)),
    ],
)
def _sc_apply(img_hbm, mask_hbm, mlut_hbm, out_hbm, lut_v, in_v, m_v, o_v,
              sem_i, sem_m, sem_o):
    wid = lax.axis_index("s") * 2 + lax.axis_index("c")
    batch = wid // _PARTS
    row0 = (wid % _PARTS) * _ROWS_PER_TEC

    pltpu.sync_copy(mlut_hbm.at[batch], lut_v)

    offs = (1, _DIM, _DIM + 1,
            _DIM * _DIM, _DIM * _DIM + 1, _DIM * _DIM + _DIM,
            _DIM * _DIM + _DIM + 1)

    def in_copies(g, slot):
        row = row0 + g * _RCH
        return (
            pltpu.make_async_copy(
                img_hbm.at[batch, :, pl.ds(row, _RCH), :],
                in_v.at[slot], sem_i.at[slot]),
            pltpu.make_async_copy(
                mask_hbm.at[batch, 0, pl.ds(row, _RCH), :],
                m_v.at[slot], sem_m.at[slot]),
        )

    def out_copy(g, slot):
        row = row0 + g * _RCH
        return (
            pltpu.make_async_copy(
                o_v.at[slot], out_hbm.at[batch, :, pl.ds(row, _RCH), :],
                sem_o.at[slot]),
        )

    def compute_row(slot, r2):
        @pl.loop(0, _W, step=16)
        def _px(i):
            sl = pl.ds(i, 16)
            rgb = (in_v[slot, 0, r2, sl], in_v[slot, 1, r2, sl],
                   in_v[slot, 2, r2, sl])
            m = m_v[slot, r2, sl]
            rs = rgb[0] * _INV_BIN
            gs = rgb[1] * _INV_BIN
            bs = rgb[2] * _INV_BIN
            ri = rs.astype(jnp.int32)
            gi = gs.astype(jnp.int32)
            bi = bs.astype(jnp.int32)
            fr = rs - ri.astype(jnp.float32)
            fg = gs - gi.astype(jnp.float32)
            fb = bs - bi.astype(jnp.float32)
            wr0 = 1.0 - fr
            wg0 = 1.0 - fg
            wb0 = 1.0 - fb
            w00 = wr0 * wg0
            w10 = fr * wg0
            w01 = wr0 * fg
            w11 = fr * fg
            w = (w00 * wb0, w10 * wb0, w01 * wb0, w11 * wb0,
                 w00 * fb, w10 * fb, w01 * fb, w11 * fb)
            idx0 = ri + gi * _DIM + bi * (_DIM * _DIM)
            idxs = (idx0,) + tuple(idx0 + o for o in offs)
            one_m = 1.0 - m
            for c in range(3):
                plane = lut_v.at[pl.ds(c * _CPAD, _CPAD)]
                v = [plsc.load_gather(plane, [ix]) for ix in idxs]
                acc = (((v[0] * w[0] + v[1] * w[1])
                        + (v[2] * w[2] + v[3] * w[3]))
                       + ((v[4] * w[4] + v[5] * w[5])
                          + (v[6] * w[6] + v[7] * w[7])))
                o_v[slot, c, r2, sl] = acc * m + rgb[c] * one_m

    for c in in_copies(0, 0) + in_copies(1, 1):
        c.start()

    @pl.loop(0, _NCHUNK, step=2)
    def _chunk(g):
        for slot in (0, 1):
            gg = g + slot
            for c in in_copies(gg, slot):
                c.wait()

            @pl.when(gg >= 2)
            def _():
                for c in out_copy(gg - 2, slot):
                    c.wait()

            for r2 in range(_RCH):
                compute_row(slot, r2)
            for c in out_copy(gg, slot):
                c.start()

            @pl.when(gg + 2 < _NCHUNK)
            def _():
                for c in in_copies(gg + 2, slot):
                    c.start()

    for c in out_copy(_NCHUNK - 2, 0) + out_copy(_NCHUNK - 1, 1):
        c.wait()


def kernel(encoder_outputs, image, mask, conv_w, conv_b, fc_w, fc_b,
           lut0, lut1, lut2, lut3, lut4, lut5):
    fea = encoder_outputs.reshape(_B * _C_IN, _FHW).astype(jnp.bfloat16)
    maskr = mask.reshape(_B, _H, _W)
    flat6 = [l.reshape(3, _N3) for l in (lut0, lut1, lut2, lut3, lut4, lut5)]

    mlut = _head(fea, maskr, jnp.asarray(_RY), jnp.asarray(_EH),
                 jnp.asarray(_RXE), conv_w, conv_b, fc_w, fc_b, *flat6)

    return _sc_apply(image, mask, mlut)


# speedup vs baseline: 1.1144x; 1.0066x over previous
import dataclasses
import functools

import numpy as np
import jax
import jax.numpy as jnp
from jax import lax
from jax.experimental import pallas as pl
from jax.experimental.pallas import tpu as pltpu
from jax.experimental.pallas import tpu_sc as plsc

_DIM = 33
_N3 = _DIM * _DIM * _DIM
_FLAT = 3 * _N3
_CPAD = 36224
_PAD = 3 * _CPAD
_B = 4
_H = _W = 512
_HW = _H * _W
_FH = _FW = 32
_FHW = _FH * _FW
_C_IN = 1024
_C_MID = 256
_NLUT = 6
_CH = 1024
_NTEC = 32
_PARTS = _NTEC // _B
_PER_TEC = _HW // _PARTS
_NCHUNK = _PER_TEC // _CH
_INV_BIN = np.float32((_DIM - 1) / 1.000001)


def _resize_mat():
    ys = np.linspace(0.0, float(_H - 1), _FH)
    y0 = np.floor(ys).astype(np.int64)
    y1 = np.clip(y0 + 1, 0, _H - 1)
    wy = (ys - y0).astype(np.float32)
    r = np.zeros((_FH, _H), np.float32)
    r[np.arange(_FH), y0] += 1.0 - wy
    r[np.arange(_FH), y1] += wy
    return r


_RY = _resize_mat()
_EH = np.kron(np.eye(_FH, dtype=np.float32),
              np.ones((_FW, 1), np.float32))
_RXE = np.tile(_RY, (_FH, 1)).astype(np.float32)

_DOT = functools.partial(jnp.dot, precision=lax.Precision.HIGHEST)


def _head_body(fea_ref, maskr_ref, ry_ref, eh_ref, rxe_ref, cw_ref, cb_ref,
               fcw_ref, fcb_ref, l0_ref, l1_ref, l2_ref, l3_ref, l4_ref,
               l5_ref, out_ref):
    ry = ry_ref[...]
    eh = eh_ref[...]
    rxe = rxe_ref[...]
    cw = cw_ref[...]
    cb = cb_ref[...].reshape(_C_MID, 1)
    fcw = fcw_ref[...]
    fcb = fcb_ref[...].reshape(_NLUT, 1)

    eh_bf = eh.astype(jnp.bfloat16)
    masks_wide = jnp.concatenate([maskr_ref[b] for b in range(_B)], axis=1)
    q_all = jnp.dot(ry.astype(jnp.bfloat16),
                    masks_wide.astype(jnp.bfloat16),
                    preferred_element_type=jnp.float32)
    dmfs = []
    for b in range(_B):
        q = q_all[:, b * _W:(b + 1) * _W]
        ehq = jnp.dot(eh_bf, q.astype(jnp.bfloat16),
                      preferred_element_type=jnp.float32)
        dmf = jnp.sum(ehq * rxe, axis=1, keepdims=True)
        dmfs.append(dmf)

    ones_col = jnp.ones((_FHW, 1), jnp.float32)
    dm5 = jnp.concatenate(dmfs + [ones_col], axis=1)
    s_all = jnp.dot(fea_ref[...], dm5.astype(jnp.bfloat16),
                    preferred_element_type=jnp.float32)

    cols = []
    cnts = []
    for b in range(_B):
        s_fg = s_all[b * _C_IN:(b + 1) * _C_IN, b:b + 1]
        s_full = s_all[b * _C_IN:(b + 1) * _C_IN, _B:_B + 1]
        s_bg = s_full - s_fg
        cols += [s_fg, s_bg, s_full]
        m_fg = jnp.sum(dmfs[b], axis=0, keepdims=True)
        m_bg = jnp.sum(1.0 - dmfs[b], axis=0, keepdims=True)
        cnts += [m_fg, m_bg, m_fg + m_bg]
    sp = jnp.concatenate(cols, axis=1)
    cnt = jnp.concatenate(cnts, axis=1)

    feat = _DOT(cw, sp)
    feat = (feat + cb * cnt) / (cnt + 1e-6)

    logits = fcb
    for j in range(3):
        feat_j = jnp.concatenate(
            [feat[:, 3 * b + j:3 * b + j + 1] for b in range(_B)], axis=1)
        logits = logits + _DOT(fcw[:, j * _C_MID:(j + 1) * _C_MID], feat_j)
    z = logits - jnp.max(logits, axis=0, keepdims=True)
    e = jnp.exp(z)
    p_all = e / jnp.sum(e, axis=0, keepdims=True)
    lrefs = (l0_ref, l1_ref, l2_ref, l3_ref, l4_ref, l5_ref)
    accs = [None] * _B
    for n in range(_NLUT):
        val = lrefs[n][...]
        for b in range(_B):
            term = val * p_all[n:n + 1, b:b + 1]
            accs[b] = term if accs[b] is None else accs[b] + term
    for b in range(_B):
        for c in range(3):
            out_ref[b:b + 1, c * _CPAD:c * _CPAD + _N3] = accs[b][c:c + 1, :]


_head = pl.pallas_call(
    _head_body,
    out_shape=jax.ShapeDtypeStruct((_B, _PAD), jnp.float32),
    compiler_params=pltpu.CompilerParams(vmem_limit_bytes=63 * 1024 * 1024),
)


_SC_PARAMS = pltpu.CompilerParams()
if "needs_layout_passes" in pltpu.CompilerParams.__dataclass_fields__:
    _SC_PARAMS = dataclasses.replace(_SC_PARAMS, needs_layout_passes=False)


_ROWS_PER_TEC = _H // _PARTS
_RCH = 2


@functools.partial(
    pl.kernel,
    out_type=jax.ShapeDtypeStruct((_B, 3, _H, _W), jnp.float32),
    mesh=plsc.VectorSubcoreMesh(core_axis_name="c", subcore_axis_name="s"),
    compiler_params=_SC_PARAMS,
    scratch_types=[
        pltpu.VMEM((_PAD,), jnp.float32),
        pltpu.VMEM((2, 3, _RCH, _W), jnp.float32),
        pltpu.VMEM((2, _RCH, _W), jnp.float32),
        pltpu.VMEM((2, 3, _RCH, _W), jnp.float32),
        pltpu.SemaphoreType.DMA((2,)),
        pltpu.SemaphoreType.DMA((2,)),
        pltpu.SemaphoreType.DMA((2,)),
    ],
)
def _sc_apply(img_hbm, mask_hbm, mlut_hbm, out_hbm, lut_v, in_v, m_v, o_v,
              sem_i, sem_m, sem_o):
    wid = lax.axis_index("s") * 2 + lax.axis_index("c")
    batch = wid // _PARTS
    row0 = (wid % _PARTS) * _ROWS_PER_TEC

    pltpu.sync_copy(mlut_hbm.at[batch], lut_v)

    offs = (1, _DIM, _DIM + 1,
            _DIM * _DIM, _DIM * _DIM + 1, _DIM * _DIM + _DIM,
            _DIM * _DIM + _DIM + 1)

    def in_copies(g, slot):
        row = row0 + g * _RCH
        return (
            pltpu.make_async_copy(
                img_hbm.at[batch, :, pl.ds(row, _RCH), :],
                in_v.at[slot], sem_i.at[slot]),
            pltpu.make_async_copy(
                mask_hbm.at[batch, 0, pl.ds(row, _RCH), :],
                m_v.at[slot], sem_m.at[slot]),
        )

    def out_copy(g, slot):
        row = row0 + g * _RCH
        return (
            pltpu.make_async_copy(
                o_v.at[slot], out_hbm.at[batch, :, pl.ds(row, _RCH), :],
                sem_o.at[slot]),
        )

    def compute_row(slot, r2):
        @pl.loop(0, _W, step=16)
        def _px(i):
            sl = pl.ds(i, 16)
            rgb = (in_v[slot, 0, r2, sl], in_v[slot, 1, r2, sl],
                   in_v[slot, 2, r2, sl])
            m = m_v[slot, r2, sl]
            rs = rgb[0] * _INV_BIN
            gs = rgb[1] * _INV_BIN
            bs = rgb[2] * _INV_BIN
            ri = rs.astype(jnp.int32)
            gi = gs.astype(jnp.int32)
            bi = bs.astype(jnp.int32)
            fr = rs - ri.astype(jnp.float32)
            fg = gs - gi.astype(jnp.float32)
            fb = bs - bi.astype(jnp.float32)
            wr0 = 1.0 - fr
            wg0 = 1.0 - fg
            wb0 = 1.0 - fb
            w00 = wr0 * wg0
            w10 = fr * wg0
            w01 = wr0 * fg
            w11 = fr * fg
            w = (w00 * wb0, w10 * wb0, w01 * wb0, w11 * wb0,
                 w00 * fb, w10 * fb, w01 * fb, w11 * fb)
            idx0 = ri + gi * _DIM + bi * (_DIM * _DIM)
            idxs = (idx0,) + tuple(idx0 + o for o in offs)
            one_m = 1.0 - m
            for c in range(3):
                plane = lut_v.at[pl.ds(c * _CPAD, _CPAD)]
                v = [plsc.load_gather(plane, [ix]) for ix in idxs]
                acc = (((v[0] * w[0] + v[1] * w[1])
                        + (v[2] * w[2] + v[3] * w[3]))
                       + ((v[4] * w[4] + v[5] * w[5])
                          + (v[6] * w[6] + v[7] * w[7])))
                o_v[slot, c, r2, sl] = acc * m + rgb[c] * one_m

    for c in in_copies(0, 0) + in_copies(1, 1):
        c.start()

    @pl.loop(0, _NCHUNK, step=2)
    def _chunk(g):
        for slot in (0, 1):
            gg = g + slot
            for c in in_copies(gg, slot):
                c.wait()

            @pl.when(gg >= 2)
            def _():
                for c in out_copy(gg - 2, slot):
                    c.wait()

            for r2 in range(_RCH):
                compute_row(slot, r2)
            for c in out_copy(gg, slot):
                c.start()

            @pl.when(gg + 2 < _NCHUNK)
            def _():
                for c in in_copies(gg + 2, slot):
                    c.start()

    for c in out_copy(_NCHUNK - 2, 0) + out_copy(_NCHUNK - 1, 1):
        c.wait()


def kernel(encoder_outputs, image, mask, conv_w, conv_b, fc_w, fc_b,
           lut0, lut1, lut2, lut3, lut4, lut5):
    fea = encoder_outputs.reshape(_B * _C_IN, _FHW).astype(jnp.bfloat16)
    maskr = mask.reshape(_B, _H, _W)
    flat6 = [l.reshape(3, _N3) for l in (lut0, lut1, lut2, lut3, lut4, lut5)]

    mlut = _head(fea, maskr, jnp.asarray(_RY), jnp.asarray(_EH),
                 jnp.asarray(_RXE), conv_w, conv_b, fc_w, fc_b, *flat6)

    return _sc_apply(image, mask, mlut)


# speedup vs baseline: 1.1185x; 1.0036x over previous
import dataclasses
import functools

import numpy as np
import jax
import jax.numpy as jnp
from jax import lax
from jax.experimental import pallas as pl
from jax.experimental.pallas import tpu as pltpu
from jax.experimental.pallas import tpu_sc as plsc

_DIM = 33
_N3 = _DIM * _DIM * _DIM
_FLAT = 3 * _N3
_CPAD = 36224
_PAD = 3 * _CPAD
_B = 4
_H = _W = 512
_HW = _H * _W
_FH = _FW = 32
_FHW = _FH * _FW
_C_IN = 1024
_C_MID = 256
_NLUT = 6
_CH = 1024
_NTEC = 32
_PARTS = _NTEC // _B
_PER_TEC = _HW // _PARTS
_NCHUNK = _PER_TEC // _CH
_INV_BIN = np.float32((_DIM - 1) / 1.000001)


def _resize_mat():
    ys = np.linspace(0.0, float(_H - 1), _FH)
    y0 = np.floor(ys).astype(np.int64)
    y1 = np.clip(y0 + 1, 0, _H - 1)
    wy = (ys - y0).astype(np.float32)
    r = np.zeros((_FH, _H), np.float32)
    r[np.arange(_FH), y0] += 1.0 - wy
    r[np.arange(_FH), y1] += wy
    return r


_RY = _resize_mat()
_EH = np.kron(np.eye(_FH, dtype=np.float32),
              np.ones((_FW, 1), np.float32))
_RXE = np.tile(_RY, (_FH, 1)).astype(np.float32)

_DOT = functools.partial(jnp.dot, precision=lax.Precision.HIGHEST)


def _head_body(fea_ref, maskr_ref, ry_ref, eh_ref, rxe_ref, cw_ref, cb_ref,
               fcw_ref, fcb_ref, l0_ref, l1_ref, l2_ref, l3_ref, l4_ref,
               l5_ref, out_ref):
    ry = ry_ref[...]
    eh = eh_ref[...]
    rxe = rxe_ref[...]
    cw = cw_ref[...]
    cb = cb_ref[...].reshape(_C_MID, 1)
    fcw = fcw_ref[...]
    fcb = fcb_ref[...].reshape(_NLUT, 1)

    eh_bf = eh.astype(jnp.bfloat16)
    masks_wide = jnp.concatenate([maskr_ref[b] for b in range(_B)], axis=1)
    q_all = jnp.dot(ry.astype(jnp.bfloat16),
                    masks_wide.astype(jnp.bfloat16),
                    preferred_element_type=jnp.float32)
    dmfs = []
    for b in range(_B):
        q = q_all[:, b * _W:(b + 1) * _W]
        ehq = jnp.dot(eh_bf, q.astype(jnp.bfloat16),
                      preferred_element_type=jnp.float32)
        dmf = jnp.sum(ehq * rxe, axis=1, keepdims=True)
        dmfs.append(dmf)

    ones_col = jnp.ones((_FHW, 1), jnp.float32)
    dm5 = jnp.concatenate(dmfs + [ones_col], axis=1)
    s_all = jnp.dot(fea_ref[...], dm5.astype(jnp.bfloat16),
                    preferred_element_type=jnp.float32)

    cols = []
    cnts = []
    for b in range(_B):
        s_fg = s_all[b * _C_IN:(b + 1) * _C_IN, b:b + 1]
        s_full = s_all[b * _C_IN:(b + 1) * _C_IN, _B:_B + 1]
        s_bg = s_full - s_fg
        cols += [s_fg, s_bg, s_full]
        m_fg = jnp.sum(dmfs[b], axis=0, keepdims=True)
        m_bg = jnp.sum(1.0 - dmfs[b], axis=0, keepdims=True)
        cnts += [m_fg, m_bg, m_fg + m_bg]
    sp = jnp.concatenate(cols, axis=1)
    cnt = jnp.concatenate(cnts, axis=1)

    feat = _DOT(cw, sp)
    feat = (feat + cb * cnt) / (cnt + 1e-6)

    logits = fcb
    for j in range(3):
        feat_j = jnp.concatenate(
            [feat[:, 3 * b + j:3 * b + j + 1] for b in range(_B)], axis=1)
        logits = logits + _DOT(fcw[:, j * _C_MID:(j + 1) * _C_MID], feat_j)
    z = logits - jnp.max(logits, axis=0, keepdims=True)
    e = jnp.exp(z)
    p_all = e / jnp.sum(e, axis=0, keepdims=True)
    lrefs = (l0_ref, l1_ref, l2_ref, l3_ref, l4_ref, l5_ref)
    accs = [None] * _B
    for n in range(_NLUT):
        val = lrefs[n][...]
        for b in range(_B):
            term = val * p_all[n:n + 1, b:b + 1]
            accs[b] = term if accs[b] is None else accs[b] + term
    for b in range(_B):
        for c in range(3):
            out_ref[b:b + 1, c * _CPAD:c * _CPAD + _N3] = accs[b][c:c + 1, :]


_head = pl.pallas_call(
    _head_body,
    out_shape=jax.ShapeDtypeStruct((_B, _PAD), jnp.float32),
    compiler_params=pltpu.CompilerParams(vmem_limit_bytes=63 * 1024 * 1024),
)


_SC_PARAMS = pltpu.CompilerParams()
if "needs_layout_passes" in pltpu.CompilerParams.__dataclass_fields__:
    _SC_PARAMS = dataclasses.replace(_SC_PARAMS, needs_layout_passes=False)


_ROWS_PER_TEC = _H // _PARTS
_RCH = 2


@functools.partial(
    pl.kernel,
    out_type=jax.ShapeDtypeStruct((_B, 3, _H, _W), jnp.float32),
    mesh=plsc.VectorSubcoreMesh(core_axis_name="c", subcore_axis_name="s"),
    compiler_params=_SC_PARAMS,
    scratch_types=[
        pltpu.VMEM((_PAD,), jnp.float32),
        pltpu.VMEM((2, 3, _RCH, _W), jnp.float32),
        pltpu.VMEM((2, _RCH, _W), jnp.float32),
        pltpu.VMEM((2, 3, _RCH, _W), jnp.float32),
        pltpu.SemaphoreType.DMA((2,)),
        pltpu.SemaphoreType.DMA((2,)),
        pltpu.SemaphoreType.DMA((2,)),
    ],
)
def _sc_apply(img_hbm, mask_hbm, mlut_hbm, out_hbm, lut_v, in_v, m_v, o_v,
              sem_i, sem_m, sem_o):
    wid = lax.axis_index("s") * 2 + lax.axis_index("c")
    batch = wid // _PARTS
    row0 = (wid % _PARTS) * _ROWS_PER_TEC

    lut_copy = pltpu.make_async_copy(mlut_hbm.at[batch], lut_v, sem_o.at[0])
    lut_copy.start()

    offs = (1, _DIM, _DIM + 1,
            _DIM * _DIM, _DIM * _DIM + 1, _DIM * _DIM + _DIM,
            _DIM * _DIM + _DIM + 1)

    def in_copies(g, slot):
        row = row0 + g * _RCH
        return (
            pltpu.make_async_copy(
                img_hbm.at[batch, :, pl.ds(row, _RCH), :],
                in_v.at[slot], sem_i.at[slot]),
            pltpu.make_async_copy(
                mask_hbm.at[batch, 0, pl.ds(row, _RCH), :],
                m_v.at[slot], sem_m.at[slot]),
        )

    def out_copy(g, slot):
        row = row0 + g * _RCH
        return (
            pltpu.make_async_copy(
                o_v.at[slot], out_hbm.at[batch, :, pl.ds(row, _RCH), :],
                sem_o.at[slot]),
        )

    def compute_row(slot, r2):
        @pl.loop(0, _W, step=16)
        def _px(i):
            sl = pl.ds(i, 16)
            rgb = (in_v[slot, 0, r2, sl], in_v[slot, 1, r2, sl],
                   in_v[slot, 2, r2, sl])
            m = m_v[slot, r2, sl]
            rs = rgb[0] * _INV_BIN
            gs = rgb[1] * _INV_BIN
            bs = rgb[2] * _INV_BIN
            ri = rs.astype(jnp.int32)
            gi = gs.astype(jnp.int32)
            bi = bs.astype(jnp.int32)
            fr = rs - ri.astype(jnp.float32)
            fg = gs - gi.astype(jnp.float32)
            fb = bs - bi.astype(jnp.float32)
            wr0 = 1.0 - fr
            wg0 = 1.0 - fg
            wb0 = 1.0 - fb
            w00 = wr0 * wg0
            w10 = fr * wg0
            w01 = wr0 * fg
            w11 = fr * fg
            w = (w00 * wb0, w10 * wb0, w01 * wb0, w11 * wb0,
                 w00 * fb, w10 * fb, w01 * fb, w11 * fb)
            idx0 = ri + gi * _DIM + bi * (_DIM * _DIM)
            idxs = (idx0,) + tuple(idx0 + o for o in offs)
            one_m = 1.0 - m
            for c in range(3):
                plane = lut_v.at[pl.ds(c * _CPAD, _CPAD)]
                v = [plsc.load_gather(plane, [ix]) for ix in idxs]
                acc = (((v[0] * w[0] + v[1] * w[1])
                        + (v[2] * w[2] + v[3] * w[3]))
                       + ((v[4] * w[4] + v[5] * w[5])
                          + (v[6] * w[6] + v[7] * w[7])))
                o_v[slot, c, r2, sl] = acc * m + rgb[c] * one_m

    for c in in_copies(0, 0) + in_copies(1, 1):
        c.start()
    lut_copy.wait()

    @pl.loop(0, _NCHUNK, step=2)
    def _chunk(g):
        for slot in (0, 1):
            gg = g + slot
            for c in in_copies(gg, slot):
                c.wait()

            @pl.when(gg >= 2)
            def _():
                for c in out_copy(gg - 2, slot):
                    c.wait()

            for r2 in range(_RCH):
                compute_row(slot, r2)
            for c in out_copy(gg, slot):
                c.start()

            @pl.when(gg + 2 < _NCHUNK)
            def _():
                for c in in_copies(gg + 2, slot):
                    c.start()

    for c in out_copy(_NCHUNK - 2, 0) + out_copy(_NCHUNK - 1, 1):
        c.wait()


def kernel(encoder_outputs, image, mask, conv_w, conv_b, fc_w, fc_b,
           lut0, lut1, lut2, lut3, lut4, lut5):
    fea = encoder_outputs.reshape(_B * _C_IN, _FHW).astype(jnp.bfloat16)
    maskr = mask.reshape(_B, _H, _W)
    flat6 = [l.reshape(3, _N3) for l in (lut0, lut1, lut2, lut3, lut4, lut5)]

    mlut = _head(fea, maskr, jnp.asarray(_RY), jnp.asarray(_EH),
                 jnp.asarray(_RXE), conv_w, conv_b, fc_w, fc_b, *flat6)

    return _sc_apply(image, mask, mlut)
